# baseline (speedup 1.0000x reference)
import jax
import jax.numpy as jnp
from jax.experimental import pallas as pl
from jax.experimental.pallas import tpu as pltpu

BN = 1000


def _add_kernel(x_ref, emb_ref, o_ref):
    o_ref[...] = x_ref[...] + emb_ref[...][None, :, None, :]


def kernel(x, emb_weight):
    batch, n, t, d = x.shape
    return pl.pallas_call(
        _add_kernel,
        grid=(batch, n // BN),
        in_specs=[
            pl.BlockSpec((1, BN, t, d), lambda b, i: (b, i, 0, 0)),
            pl.BlockSpec((BN, d), lambda b, i: (i, 0)),
        ],
        out_specs=pl.BlockSpec((1, BN, t, d), lambda b, i: (b, i, 0, 0)),
        out_shape=jax.ShapeDtypeStruct((batch, n, t, d), x.dtype),
        compiler_params=pltpu.CompilerParams(
            dimension_semantics=("parallel", "parallel"),
        ),
    )(x, emb_weight)
